# baseline (speedup 1.0000x reference)
import functools

import jax
import jax.numpy as jnp
from jax import lax
from jax.experimental import pallas as pl
from jax.experimental.pallas import tpu as pltpu
from jax.experimental.pallas import tpu_sc as plsc

L = 16
NSC = 2
NT = 16
K = 128


def _table_body(x_ref, w_ref, out_ref):
    out_ref[0] = jnp.dot(x_ref[...], w_ref[0],
                         preferred_element_type=jnp.float32)


def _make_table(x, w, block=1000):
    n, d = x.shape
    r, _, h = w.shape
    wj = jnp.transpose(w, (2, 1, 0))
    return pl.pallas_call(
        _table_body,
        grid=(h, n // block),
        in_specs=[
            pl.BlockSpec((block, d), lambda j, i: (i, 0)),
            pl.BlockSpec((1, d, r), lambda j, i: (j, 0, 0)),
        ],
        out_specs=pl.BlockSpec((1, block, r), lambda j, i: (j, i, 0)),
        out_shape=jax.ShapeDtypeStruct((h, n, r), jnp.float32),
    )(x, wj)


def _combine_body(relu, x_ref, root_ref, b_ref, agg_ref, out_ref):
    o = jnp.dot(x_ref[...], root_ref[...],
                preferred_element_type=jnp.float32)
    o = o + b_ref[...] + agg_ref[...]
    if relu:
        o = jnp.maximum(o, 0.0)
    out_ref[...] = o


def _combine(x, root, b, agg, relu, block=1000):
    n, d = x.shape
    h = root.shape[1]
    return pl.pallas_call(
        functools.partial(_combine_body, relu),
        grid=(n // block,),
        in_specs=[
            pl.BlockSpec((block, d), lambda i: (i, 0)),
            pl.BlockSpec((d, h), lambda i: (0, 0)),
            pl.BlockSpec((1, h), lambda i: (0, 0)),
            pl.BlockSpec((block, h), lambda i: (i, 0)),
        ],
        out_specs=pl.BlockSpec((block, h), lambda i: (i, 0)),
        out_shape=jax.ShapeDtypeStruct((n, h), jnp.float32),
    )(x, root, b.reshape(1, h), agg)


def _weights_call(n, r, e_pad, dst, etype):
    n_half = n // NSC
    n_q = n_half // 2
    c_q = n_q * r
    c_pad = (c_q // (NT * K) + 1) * (NT * K)
    e_tile = e_pad // NT
    n_chunks = e_tile // K
    z_per = c_pad // NT

    mesh = plsc.VectorSubcoreMesh(core_axis_name="c", subcore_axis_name="s",
                                  num_cores=NSC, num_subcores=NT)

    @functools.partial(
        pl.kernel,
        out_type=jax.ShapeDtypeStruct((e_pad + 8,), jnp.float32),
        mesh=mesh,
        scratch_types=[
            pltpu.VMEM((K,), jnp.int32),
            pltpu.VMEM((K,), jnp.int32),
            pltpu.VMEM((K,), jnp.int32),
            pltpu.VMEM((K,), jnp.int32),
            pltpu.VMEM((K,), jnp.float32),
            pltpu.VMEM((K,), jnp.float32),
            pltpu.VMEM((K,), jnp.float32),
            pltpu.VMEM_SHARED((c_pad,), jnp.float32),
            pltpu.SemaphoreType.DMA,
        ],
    )
    def k(dst_hbm, type_hbm, wgt_hbm,
          dbuf, tbuf, keybuf, pbuf, fbuf, cbuf, wbuf, csp, sem):
        cid = lax.axis_index("c")
        sid = lax.axis_index("s")
        iota = lax.iota(jnp.int32, L)

        def fill(i, val):
            fbuf[pl.ds(i * L, L)] = jnp.full((L,), val, jnp.float32)
            return val

        for q in range(2):
            base_node = cid * n_half + q * n_q

            lax.fori_loop(0, K // L, fill, 0.0)

            def zbody(i, _):
                pltpu.sync_copy(fbuf, csp.at[pl.ds(sid * z_per + i * K, K)])
                return 0

            lax.fori_loop(0, z_per // K, zbody, 0)
            lax.fori_loop(0, K // L, fill, 1.0)
            plsc.subcore_barrier()

            def keys(g, _):
                d = dbuf[pl.ds(g * L, L)]
                t = tbuf[pl.ds(g * L, L)]
                rel = d - base_node
                key = rel * r + t
                ok = (rel >= 0) & (rel < n_q)
                keybuf[pl.ds(g * L, L)] = jnp.where(ok, key, c_q)
                return 0

            def count_chunk(i, _):
                off = sid * e_tile + i * K
                pltpu.sync_copy(dst_hbm.at[pl.ds(off, K)], dbuf)
                pltpu.sync_copy(type_hbm.at[pl.ds(off, K)], tbuf)
                lax.fori_loop(0, K // L, keys, 0)
                pltpu.sync_copy(fbuf, csp.at[keybuf], add=True)
                return 0

            lax.fori_loop(0, n_chunks, count_chunk, 0)
            plsc.subcore_barrier()

            def weight_chunk(i, _):
                off = sid * e_tile + i * K
                pltpu.sync_copy(dst_hbm.at[pl.ds(off, K)], dbuf)
                pltpu.sync_copy(type_hbm.at[pl.ds(off, K)], tbuf)
                lax.fori_loop(0, K // L, keys, 0)
                pltpu.async_copy(csp.at[keybuf], cbuf, sem).wait()

                def wv(g, _):
                    d = dbuf[pl.ds(g * L, L)]
                    rel = d - base_node
                    ok = (rel >= 0) & (rel < n_q)
                    pos = off + g * L + iota
                    pbuf[pl.ds(g * L, L)] = jnp.where(ok, pos, e_pad)
                    cv = cbuf[pl.ds(g * L, L)]
                    wbuf[pl.ds(g * L, L)] = 1.0 / jnp.maximum(cv, 1.0)
                    return 0

                lax.fori_loop(0, K // L, wv, 0)
                pltpu.sync_copy(wbuf, wgt_hbm.at[pbuf])
                return 0

            lax.fori_loop(0, n_chunks, weight_chunk, 0)
            plsc.subcore_barrier()

    return k(dst, etype)


def _layer_call(n, r, e_pad, src, dst, etype, wgt, tab):
    e_tile = e_pad // NT
    n_chunks = e_tile // K
    n2 = n // 2
    B = n * r
    acc = n2 * L
    a_pad = (acc // (NT * K) + 1) * (NT * K)
    z_per = a_pad // NT

    mesh = plsc.VectorSubcoreMesh(core_axis_name="c", subcore_axis_name="s",
                                  num_cores=NSC, num_subcores=NT)

    @functools.partial(
        pl.kernel,
        out_type=jax.ShapeDtypeStruct((NSC * 2 * a_pad,), jnp.float32),
        mesh=mesh,
        scratch_types=[
            pltpu.VMEM((K,), jnp.int32),
            pltpu.VMEM((K,), jnp.int32),
            pltpu.VMEM((K,), jnp.int32),
            pltpu.VMEM((K,), jnp.int32),
            pltpu.VMEM((K,), jnp.int32),
            pltpu.VMEM((K,), jnp.int32),
            pltpu.VMEM((K,), jnp.int32),
            pltpu.VMEM((K,), jnp.float32),
            pltpu.VMEM((K,), jnp.float32),
            pltpu.VMEM((K,), jnp.float32),
            pltpu.VMEM_SHARED((a_pad,), jnp.float32),
            pltpu.SemaphoreType.DMA,
        ],
    )
    def k(src_hbm, dst_hbm, type_hbm, wgt_hbm, tab_hbm, out_hbm,
          sbuf, dbuf, tbuf, bbuf, ob0, ibuf, obuf, wv, colv, zv, osp, sem):
        cid = lax.axis_index("c")
        sid = lax.axis_index("s")
        jbase = cid * L

        for g in range(K // L):
            zv[pl.ds(g * L, L)] = jnp.zeros((L,), jnp.float32)

        for q in range(2):
            base = q * n2

            def zbody(i, _):
                pltpu.sync_copy(zv, osp.at[pl.ds(sid * z_per + i * K, K)])
                return 0

            lax.fori_loop(0, z_per // K, zbody, 0)
            plsc.subcore_barrier()

            def chunk(i, _):
                off = sid * e_tile + i * K
                pltpu.sync_copy(src_hbm.at[pl.ds(off, K)], sbuf)
                pltpu.sync_copy(dst_hbm.at[pl.ds(off, K)], dbuf)
                pltpu.sync_copy(type_hbm.at[pl.ds(off, K)], tbuf)
                pltpu.sync_copy(wgt_hbm.at[pl.ds(off, K)], wv)

                def gbody(g, _):
                    s = sbuf[pl.ds(g * L, L)]
                    t = tbuf[pl.ds(g * L, L)]
                    d = dbuf[pl.ds(g * L, L)]
                    bbuf[pl.ds(g * L, L)] = s * r + t
                    loc = d - base
                    ok = (loc >= 0) & (loc < n2)
                    ob0[pl.ds(g * L, L)] = jnp.where(ok, loc * L, acc)
                    return 0

                lax.fori_loop(0, K // L, gbody, 0)

                for j in range(L):
                    for g in range(K // L):
                        sl = pl.ds(g * L, L)
                        ibuf[sl] = bbuf[sl] + (jbase + j) * B
                        obuf[sl] = ob0[sl] + j
                    pltpu.async_copy(tab_hbm.at[ibuf], colv, sem).wait()
                    for g in range(K // L):
                        sl = pl.ds(g * L, L)
                        colv[sl] = colv[sl] * wv[sl]
                    pltpu.sync_copy(colv, osp.at[obuf], add=True)
                return 0

            lax.fori_loop(0, n_chunks, chunk, 0)
            plsc.subcore_barrier()

            def drain(i, _):
                a = sid * z_per + i * K
                pltpu.sync_copy(osp.at[pl.ds(a, K)], colv)
                pltpu.sync_copy(colv,
                                out_hbm.at[pl.ds((cid * 2 + q) * a_pad + a, K)])
                return 0

            lax.fori_loop(0, z_per // K, drain, 0)
            plsc.subcore_barrier()

    return k(src, dst, etype, wgt, tab)


def kernel(node_emb, W1, root1, b1, W2, root2, b2, edge_index, edge_type):
    n, d = node_emb.shape
    r, _, h = W1.shape
    e = edge_type.shape[0]
    src = edge_index[0].astype(jnp.int32)
    dst = edge_index[1].astype(jnp.int32)
    et = edge_type.astype(jnp.int32)

    quant = NT * K
    e_pad = ((e + quant - 1) // quant) * quant
    pad = e_pad - e
    if pad:
        src = jnp.concatenate([src, jnp.zeros((pad,), jnp.int32)])
        dst = jnp.concatenate([dst, jnp.full((pad,), n, jnp.int32)])
        et = jnp.concatenate([et, jnp.zeros((pad,), jnp.int32)])

    wgt = _weights_call(n, r, e_pad, dst, et)

    def layer(x, W, root, b, relu):
        h = W.shape[2]
        tab = _make_table(x, W).reshape(h * n * r)
        out1 = _layer_call(n, r, e_pad, src, dst, et, wgt, tab)
        n2 = n // 2
        acc = n2 * L
        a_pad = (acc // (NT * K) + 1) * (NT * K)

        def part(c, q):
            o = (c * 2 + q) * a_pad
            return out1[o:o + acc].reshape(n2, L)

        agg_full = jnp.concatenate(
            [jnp.concatenate([part(c, 0), part(c, 1)], axis=0)
             for c in range(2)], axis=1)
        return _combine(x, root, b, agg_full, relu)

    h1 = layer(node_emb, W1, root1, b1, True)
    return layer(h1, W2, root2, b2, False)
